# baseline (speedup 1.0000x reference)
import functools

import jax
import jax.numpy as jnp
from jax import lax
from jax.experimental import pallas as pl
from jax.experimental.pallas import tpu as pltpu
from jax.experimental.pallas import tpu_sc as plsc

B = 16384
D = 16
NROWS = 1000000
NC = 2
NS = 16
NW = NC * NS
ROWS_PER_W = B // NW
CHUNK = 128
NCHUNK = ROWS_PER_W // CHUNK

TBLK = 8192


def _transpose_body(tu, ti, eye, ou, oi):
    dn = (((0,), (0,)), ((), ()))
    ou[...] = lax.dot_general(tu[...], eye[...], dn,
                              preferred_element_type=jnp.float32)
    oi[...] = lax.dot_general(ti[...], eye[...], dn,
                              preferred_element_type=jnp.float32)


def _transpose_tables(tu, ti):
    eye = jnp.eye(D, dtype=jnp.float32)
    grid = (pl.cdiv(NROWS, TBLK),)
    return pl.pallas_call(
        _transpose_body,
        grid=grid,
        in_specs=[
            pl.BlockSpec((D, TBLK), lambda i: (0, i)),
            pl.BlockSpec((D, TBLK), lambda i: (0, i)),
            pl.BlockSpec((D, D), lambda i: (0, 0)),
        ],
        out_specs=[
            pl.BlockSpec((TBLK, D), lambda i: (i, 0)),
            pl.BlockSpec((TBLK, D), lambda i: (i, 0)),
        ],
        out_shape=[
            jax.ShapeDtypeStruct((NROWS, D), jnp.float32),
            jax.ShapeDtypeStruct((NROWS, D), jnp.float32),
        ],
    )(tu, ti, eye)


def _sc_gather_body(uid, iid, uemb, iemb, gu, gi, idx_v, rows_v, sem):
    wid = lax.axis_index("s") * NC + lax.axis_index("c")
    base = wid * ROWS_PER_W
    for j in range(NCHUNK):
        pltpu.sync_copy(uid.at[pl.ds(base + j * CHUNK, CHUNK)], idx_v.at[j])
        pltpu.sync_copy(iid.at[pl.ds(base + j * CHUNK, CHUNK)],
                        idx_v.at[NCHUNK + j])
    copies = []
    for j in range(NCHUNK):
        copies.append(pltpu.async_copy(uemb.at[idx_v.at[j]], rows_v.at[j], sem))
    for j in range(NCHUNK):
        copies.append(
            pltpu.async_copy(iemb.at[idx_v.at[NCHUNK + j]],
                             rows_v.at[NCHUNK + j], sem))
    for c in copies:
        c.wait()
    for j in range(NCHUNK):
        pltpu.sync_copy(rows_v.at[j], gu.at[pl.ds(base + j * CHUNK, CHUNK)])
        pltpu.sync_copy(rows_v.at[NCHUNK + j],
                        gi.at[pl.ds(base + j * CHUNK, CHUNK)])


def _sc_gather(uid, iid, uemb, iemb):
    mesh = plsc.VectorSubcoreMesh(core_axis_name="c", subcore_axis_name="s")
    return pl.kernel(
        _sc_gather_body,
        out_type=(
            jax.ShapeDtypeStruct((B, D), jnp.float32),
            jax.ShapeDtypeStruct((B, D), jnp.float32),
        ),
        mesh=mesh,
        scratch_types=[
            pltpu.VMEM((2 * NCHUNK, CHUNK), jnp.int32),
            pltpu.VMEM((2 * NCHUNK, CHUNK, D), jnp.float32),
            pltpu.SemaphoreType.DMA,
        ],
        compiler_params=pltpu.CompilerParams(use_tc_tiling_on_sc=False),
    )(uid, iid, uemb, iemb)


BLK = 2048


def _mlp_body(gu, gi, w1u, w1i, b1, w2, b2, w3, b3, out):
    h = jnp.dot(gu[...], w1u[...], preferred_element_type=jnp.float32)
    h = h + jnp.dot(gi[...], w1i[...], preferred_element_type=jnp.float32)
    h = jnp.maximum(h + b1[...], 0.0)
    h = jnp.maximum(
        jnp.dot(h, w2[...], preferred_element_type=jnp.float32) + b2[...], 0.0)
    out[...] = jnp.dot(h, w3[...], preferred_element_type=jnp.float32) + b3[...]


def _mlp(gu, gi, W1, b1, W2, b2, W3, b3):
    w1u = W1[:D, :]
    w1i = W1[D:, :]
    b1r = jnp.reshape(b1, (1, -1))
    b2r = jnp.reshape(b2, (1, -1))
    b3r = jnp.reshape(b3, (1, -1))
    grid = (B // BLK,)
    return pl.pallas_call(
        _mlp_body,
        grid=grid,
        in_specs=[
            pl.BlockSpec((BLK, D), lambda i: (i, 0)),
            pl.BlockSpec((BLK, D), lambda i: (i, 0)),
            pl.BlockSpec(w1u.shape, lambda i: (0, 0)),
            pl.BlockSpec(w1i.shape, lambda i: (0, 0)),
            pl.BlockSpec(b1r.shape, lambda i: (0, 0)),
            pl.BlockSpec(W2.shape, lambda i: (0, 0)),
            pl.BlockSpec(b2r.shape, lambda i: (0, 0)),
            pl.BlockSpec(W3.shape, lambda i: (0, 0)),
            pl.BlockSpec(b3r.shape, lambda i: (0, 0)),
        ],
        out_specs=pl.BlockSpec((BLK, 1), lambda i: (i, 0)),
        out_shape=jax.ShapeDtypeStruct((B, 1), jnp.float32),
    )(gu, gi, w1u, w1i, b1r, W2, b2r, W3, b3r)


def kernel(user_id, item_id, user_emb, item_emb, W1, b1, W2, b2, W3, b3):
    uemb_r, iemb_r = _transpose_tables(user_emb.T, item_emb.T)
    gu, gi = _sc_gather(user_id.astype(jnp.int32), item_id.astype(jnp.int32),
                        uemb_r, iemb_r)
    return _mlp(gu, gi, W1, b1, W2, b2, W3, b3)


# speedup vs baseline: 5.4725x; 5.4725x over previous
import functools

import jax
import jax.numpy as jnp
from jax import lax
from jax.experimental import pallas as pl
from jax.experimental.pallas import tpu as pltpu
from jax.experimental.pallas import tpu_sc as plsc

B = 16384
D = 16
NROWS = 1000000
LANE = 128
NC = 2
NS = 16
NW = NC * NS
ROWS_PER_W = B // NW
GRP = 16
NG = ROWS_PER_W // GRP


def _sc_gather_body(uid, iid, uemb_t, iemb_t, gu_t, gi_t,
                    idx_v, slab, out_v, sem):
    wid = lax.axis_index("s") * NC + lax.axis_index("c")
    base = wid * ROWS_PER_W
    iota = lax.iota(jnp.int32, D)

    for ids, tab, out in ((uid, uemb_t, gu_t), (iid, iemb_t, gi_t)):
        pltpu.sync_copy(ids.at[pl.ds(base, ROWS_PER_W)], idx_v)

        def grp_body(g, carry, tab=tab):
            iv = idx_v[pl.ds(g * GRP, GRP)]
            cv = lax.shift_right_logical(iv, 7)
            lv = lax.bitwise_and(iv, LANE - 1)
            copies = []
            for b in range(GRP):
                copies.append(
                    pltpu.async_copy(tab.at[:, pl.ds(cv[b] * LANE, LANE)],
                                     slab.at[b], sem))
            for cp in copies:
                cp.wait()
            for b in range(GRP):
                vec = plsc.load_gather(
                    slab.at[b], [iota, jnp.full((D,), lv[b], jnp.int32)])
                plsc.store_scatter(
                    out_v,
                    [iota, jnp.full((D,), g * GRP + b, jnp.int32)], vec)
            return carry

        lax.fori_loop(0, NG, grp_body, 0)
        pltpu.sync_copy(out_v, out.at[:, pl.ds(base, ROWS_PER_W)])


@jax.jit
def _sc_gather(uid, iid, uemb_t, iemb_t):
    mesh = plsc.VectorSubcoreMesh(core_axis_name="c", subcore_axis_name="s")
    return pl.kernel(
        _sc_gather_body,
        out_type=(
            jax.ShapeDtypeStruct((D, B), jnp.float32),
            jax.ShapeDtypeStruct((D, B), jnp.float32),
        ),
        mesh=mesh,
        scratch_types=[
            pltpu.VMEM((ROWS_PER_W,), jnp.int32),
            pltpu.VMEM((GRP, D, LANE), jnp.float32),
            pltpu.VMEM((D, ROWS_PER_W), jnp.float32),
            pltpu.SemaphoreType.DMA,
        ],
        compiler_params=pltpu.CompilerParams(needs_layout_passes=False),
    )(uid, iid, uemb_t, iemb_t)


BLK = 2048


def _mlp_body(gu_t, gi_t, w1u, w1i, b1, w2, b2, w3, b3, out):
    dn = (((0,), (0,)), ((), ()))
    h = lax.dot_general(gu_t[...], w1u[...], dn,
                        preferred_element_type=jnp.float32)
    h = h + lax.dot_general(gi_t[...], w1i[...], dn,
                            preferred_element_type=jnp.float32)
    h = jnp.maximum(h + b1[...], 0.0)
    h = jnp.maximum(
        jnp.dot(h, w2[...], preferred_element_type=jnp.float32) + b2[...], 0.0)
    out[...] = jnp.dot(h, w3[...], preferred_element_type=jnp.float32) + b3[...]


def _mlp(gu_t, gi_t, W1, b1, W2, b2, W3, b3):
    w1u = W1[:D, :]
    w1i = W1[D:, :]
    b1r = jnp.reshape(b1, (1, -1))
    b2r = jnp.reshape(b2, (1, -1))
    b3r = jnp.reshape(b3, (1, -1))
    grid = (B // BLK,)
    return pl.pallas_call(
        _mlp_body,
        grid=grid,
        in_specs=[
            pl.BlockSpec((D, BLK), lambda i: (0, i)),
            pl.BlockSpec((D, BLK), lambda i: (0, i)),
            pl.BlockSpec(w1u.shape, lambda i: (0, 0)),
            pl.BlockSpec(w1i.shape, lambda i: (0, 0)),
            pl.BlockSpec(b1r.shape, lambda i: (0, 0)),
            pl.BlockSpec(W2.shape, lambda i: (0, 0)),
            pl.BlockSpec(b2r.shape, lambda i: (0, 0)),
            pl.BlockSpec(W3.shape, lambda i: (0, 0)),
            pl.BlockSpec(b3r.shape, lambda i: (0, 0)),
        ],
        out_specs=pl.BlockSpec((BLK, 1), lambda i: (i, 0)),
        out_shape=jax.ShapeDtypeStruct((B, 1), jnp.float32),
    )(gu_t, gi_t, w1u, w1i, b1r, W2, b2r, W3, b3r)


def kernel(user_id, item_id, user_emb, item_emb, W1, b1, W2, b2, W3, b3):
    gu_t, gi_t = _sc_gather(user_id.astype(jnp.int32),
                            item_id.astype(jnp.int32),
                            user_emb.T, item_emb.T)
    return _mlp(gu_t, gi_t, W1, b1, W2, b2, W3, b3)


# speedup vs baseline: 6.7286x; 1.2295x over previous
import functools

import jax
import jax.numpy as jnp
from jax import lax
from jax.experimental import pallas as pl
from jax.experimental.pallas import tpu as pltpu
from jax.experimental.pallas import tpu_sc as plsc

B = 16384
D = 16
NROWS = 1000000
LANE = 128
NC = 2
NS = 16
NW = NC * NS
ROWS_PER_W = B // NW
GRP = 16
NG = ROWS_PER_W // GRP


def _sc_gather_body(uid, iid, uemb_t, iemb_t, gu_t, gi_t,
                    idx_u, idx_i, slab, out_u, out_i, sem_a, sem_b):
    wid = lax.axis_index("s") * NC + lax.axis_index("c")
    base = wid * ROWS_PER_W
    iota = lax.iota(jnp.int32, D)

    pltpu.sync_copy(uid.at[pl.ds(base, ROWS_PER_W)], idx_u)
    pltpu.sync_copy(iid.at[pl.ds(base, ROWS_PER_W)], idx_i)

    def fire(idx, tab, p, g, sem):
        iv = idx[pl.ds(g * GRP, GRP)]
        cv = lax.shift_right_logical(iv, 7)
        for b in range(GRP):
            pltpu.async_copy(tab.at[:, pl.ds(cv[b] * LANE, LANE)],
                             slab.at[p, b], sem)

    def drain(tab, p, sem):
        for b in range(GRP):
            pltpu.make_async_copy(tab.at[:, pl.ds(0, LANE)],
                                  slab.at[p, b], sem).wait()

    def extract(idx, p, g, out_v):
        iv = idx[pl.ds(g * GRP, GRP)]
        lv = lax.bitwise_and(iv, LANE - 1)
        for b in range(GRP):
            vec = plsc.load_gather(
                slab.at[p, b], [iota, jnp.full((D,), lv[b], jnp.int32)])
            plsc.store_scatter(
                out_v, [iota, jnp.full((D,), g * GRP + b, jnp.int32)], vec)

    fire(idx_u, uemb_t, 0, 0, sem_a)

    def grp_body(g, carry):
        fire(idx_i, iemb_t, 1, g, sem_b)
        drain(uemb_t, 0, sem_a)
        extract(idx_u, 0, g, out_u)

        @pl.when(g + 1 < NG)
        def _():
            fire(idx_u, uemb_t, 0, g + 1, sem_a)

        drain(iemb_t, 1, sem_b)
        extract(idx_i, 1, g, out_i)
        return carry

    lax.fori_loop(0, NG, grp_body, 0)
    pltpu.sync_copy(out_u, gu_t.at[:, pl.ds(base, ROWS_PER_W)])
    pltpu.sync_copy(out_i, gi_t.at[:, pl.ds(base, ROWS_PER_W)])


@jax.jit
def _sc_gather(uid, iid, uemb_t, iemb_t):
    mesh = plsc.VectorSubcoreMesh(core_axis_name="c", subcore_axis_name="s")
    return pl.kernel(
        _sc_gather_body,
        out_type=(
            jax.ShapeDtypeStruct((D, B), jnp.float32),
            jax.ShapeDtypeStruct((D, B), jnp.float32),
        ),
        mesh=mesh,
        scratch_types=[
            pltpu.VMEM((ROWS_PER_W,), jnp.int32),
            pltpu.VMEM((ROWS_PER_W,), jnp.int32),
            pltpu.VMEM((2, GRP, D, LANE), jnp.float32),
            pltpu.VMEM((D, ROWS_PER_W), jnp.float32),
            pltpu.VMEM((D, ROWS_PER_W), jnp.float32),
            pltpu.SemaphoreType.DMA,
            pltpu.SemaphoreType.DMA,
        ],
        compiler_params=pltpu.CompilerParams(needs_layout_passes=False),
    )(uid, iid, uemb_t, iemb_t)


BLK = 2048


def _mlp_body(gu_t, gi_t, w1u, w1i, b1, w2, b2, w3, b3, out):
    dn = (((0,), (0,)), ((), ()))
    h = lax.dot_general(gu_t[...], w1u[...], dn,
                        preferred_element_type=jnp.float32)
    h = h + lax.dot_general(gi_t[...], w1i[...], dn,
                            preferred_element_type=jnp.float32)
    h = jnp.maximum(h + b1[...], 0.0)
    h = jnp.maximum(
        jnp.dot(h, w2[...], preferred_element_type=jnp.float32) + b2[...], 0.0)
    out[...] = jnp.dot(h, w3[...], preferred_element_type=jnp.float32) + b3[...]


def _mlp(gu_t, gi_t, W1, b1, W2, b2, W3, b3):
    w1u = W1[:D, :]
    w1i = W1[D:, :]
    b1r = jnp.reshape(b1, (1, -1))
    b2r = jnp.reshape(b2, (1, -1))
    b3r = jnp.reshape(b3, (1, -1))
    grid = (B // BLK,)
    return pl.pallas_call(
        _mlp_body,
        grid=grid,
        in_specs=[
            pl.BlockSpec((D, BLK), lambda i: (0, i)),
            pl.BlockSpec((D, BLK), lambda i: (0, i)),
            pl.BlockSpec(w1u.shape, lambda i: (0, 0)),
            pl.BlockSpec(w1i.shape, lambda i: (0, 0)),
            pl.BlockSpec(b1r.shape, lambda i: (0, 0)),
            pl.BlockSpec(W2.shape, lambda i: (0, 0)),
            pl.BlockSpec(b2r.shape, lambda i: (0, 0)),
            pl.BlockSpec(W3.shape, lambda i: (0, 0)),
            pl.BlockSpec(b3r.shape, lambda i: (0, 0)),
        ],
        out_specs=pl.BlockSpec((BLK, 1), lambda i: (i, 0)),
        out_shape=jax.ShapeDtypeStruct((B, 1), jnp.float32),
    )(gu_t, gi_t, w1u, w1i, b1r, W2, b2r, W3, b3r)


def kernel(user_id, item_id, user_emb, item_emb, W1, b1, W2, b2, W3, b3):
    gu_t, gi_t = _sc_gather(user_id.astype(jnp.int32),
                            item_id.astype(jnp.int32),
                            user_emb.T, item_emb.T)
    return _mlp(gu_t, gi_t, W1, b1, W2, b2, W3, b3)


# speedup vs baseline: 7.3379x; 1.0906x over previous
import functools

import jax
import jax.numpy as jnp
from jax import lax
from jax.experimental import pallas as pl
from jax.experimental.pallas import tpu as pltpu
from jax.experimental.pallas import tpu_sc as plsc

B = 16384
D = 16
NROWS = 1000000
LANE = 128
NC = 2
NS = 16
NW = NC * NS
ROWS_PER_W = B // NW
GRP = 16
NG = ROWS_PER_W // GRP


def _sc_gather_body(uid, iid, uemb_t, iemb_t, gu_t, gi_t,
                    idx_u, idx_i, slab, out_u, out_i, sem_a, sem_b):
    wid = lax.axis_index("s") * NC + lax.axis_index("c")
    base = wid * ROWS_PER_W
    iota = lax.iota(jnp.int32, D)

    pltpu.sync_copy(uid.at[pl.ds(base, ROWS_PER_W)], idx_u)
    pltpu.sync_copy(iid.at[pl.ds(base, ROWS_PER_W)], idx_i)

    def fire(idx, tab, p, g, sem):
        iv = idx[pl.ds(g * GRP, GRP)]
        cv = lax.shift_right_logical(iv, 7)
        for b in range(GRP):
            pltpu.async_copy(tab.at[:, pl.ds(cv[b] * LANE, LANE)],
                             slab.at[p, b], sem)

    def drain(tab, p, sem):
        for b in range(GRP):
            pltpu.make_async_copy(tab.at[:, pl.ds(0, LANE)],
                                  slab.at[p, b], sem).wait()

    def extract(idx, p, g, out_v):
        iv = idx[pl.ds(g * GRP, GRP)]
        lv = lax.bitwise_and(iv, LANE - 1)
        for b in range(GRP):
            vec = plsc.load_gather(
                slab.at[p, b], [iota, jnp.full((D,), lv[b], jnp.int32)])
            plsc.store_scatter(
                out_v, [iota, jnp.full((D,), g * GRP + b, jnp.int32)], vec)

    fire(idx_u, uemb_t, 0, 0, sem_a)

    def grp_body(g, carry):
        fire(idx_i, iemb_t, 1, g, sem_b)
        drain(uemb_t, 0, sem_a)
        extract(idx_u, 0, g, out_u)

        @pl.when(g + 1 < NG)
        def _():
            fire(idx_u, uemb_t, 0, g + 1, sem_a)

        drain(iemb_t, 1, sem_b)
        extract(idx_i, 1, g, out_i)
        return carry

    lax.fori_loop(0, NG, grp_body, 0)
    pltpu.sync_copy(out_u, gu_t.at[:, pl.ds(base, ROWS_PER_W)])
    pltpu.sync_copy(out_i, gi_t.at[:, pl.ds(base, ROWS_PER_W)])


@jax.jit
def _sc_gather(uid, iid, uemb_t, iemb_t):
    mesh = plsc.VectorSubcoreMesh(core_axis_name="c", subcore_axis_name="s")
    return pl.kernel(
        _sc_gather_body,
        out_type=(
            jax.ShapeDtypeStruct((D, B), jnp.float32),
            jax.ShapeDtypeStruct((D, B), jnp.float32),
        ),
        mesh=mesh,
        scratch_types=[
            pltpu.VMEM((ROWS_PER_W,), jnp.int32),
            pltpu.VMEM((ROWS_PER_W,), jnp.int32),
            pltpu.VMEM((2, GRP, D, LANE), jnp.float32),
            pltpu.VMEM((D, ROWS_PER_W), jnp.float32),
            pltpu.VMEM((D, ROWS_PER_W), jnp.float32),
            pltpu.SemaphoreType.DMA,
            pltpu.SemaphoreType.DMA,
        ],
        compiler_params=pltpu.CompilerParams(needs_layout_passes=False),
    )(uid, iid, uemb_t, iemb_t)


BLK = 4096


def _mlp_body(gu_t, gi_t, w1u, w1i, b1, w2, b2, w3, b3, out):
    dn = (((0,), (0,)), ((), ()))
    h = lax.dot_general(w1u[...], gu_t[...], dn,
                        preferred_element_type=jnp.float32)
    h = h + lax.dot_general(w1i[...], gi_t[...], dn,
                            preferred_element_type=jnp.float32)
    h = jnp.maximum(h + b1[...], 0.0)
    h = jnp.maximum(
        lax.dot_general(w2[...], h, dn, preferred_element_type=jnp.float32)
        + b2[...], 0.0)
    o = lax.dot_general(w3[...], h, dn,
                        preferred_element_type=jnp.float32) + b3[...]
    out[...] = jnp.reshape(o, (BLK,))


def _mlp(gu_t, gi_t, W1, b1, W2, b2, W3, b3):
    w1u = W1[:D, :]
    w1i = W1[D:, :]
    b1r = jnp.reshape(b1, (-1, 1))
    b2r = jnp.reshape(b2, (-1, 1))
    b3r = jnp.reshape(b3, (1, 1))
    grid = (B // BLK,)
    flat = pl.pallas_call(
        _mlp_body,
        grid=grid,
        in_specs=[
            pl.BlockSpec((D, BLK), lambda i: (0, i)),
            pl.BlockSpec((D, BLK), lambda i: (0, i)),
            pl.BlockSpec(w1u.shape, lambda i: (0, 0)),
            pl.BlockSpec(w1i.shape, lambda i: (0, 0)),
            pl.BlockSpec(b1r.shape, lambda i: (0, 0)),
            pl.BlockSpec(W2.shape, lambda i: (0, 0)),
            pl.BlockSpec(b2r.shape, lambda i: (0, 0)),
            pl.BlockSpec(W3.shape, lambda i: (0, 0)),
            pl.BlockSpec(b3r.shape, lambda i: (0, 0)),
        ],
        out_specs=pl.BlockSpec((BLK,), lambda i: (i,)),
        out_shape=jax.ShapeDtypeStruct((B,), jnp.float32),
    )(gu_t, gi_t, w1u, w1i, b1r, W2, b2r, W3, b3r)
    return jnp.reshape(flat, (B, 1))


def kernel(user_id, item_id, user_emb, item_emb, W1, b1, W2, b2, W3, b3):
    gu_t, gi_t = _sc_gather(user_id.astype(jnp.int32),
                            item_id.astype(jnp.int32),
                            user_emb.T, item_emb.T)
    return _mlp(gu_t, gi_t, W1, b1, W2, b2, W3, b3)


# speedup vs baseline: 7.7710x; 1.0590x over previous
import functools

import jax
import jax.numpy as jnp
from jax import lax
from jax.experimental import pallas as pl
from jax.experimental.pallas import tpu as pltpu
from jax.experimental.pallas import tpu_sc as plsc

B = 16384
D = 16
NROWS = 1000000
LANE = 128
NC = 2
NS = 16
NW = NC * NS
ROWS_PER_W = B // NW
GRP = 16
NG = ROWS_PER_W // GRP
NSLOT = 3


def _sc_gather_body(uid, iid, uemb_t, iemb_t, gu_t, gi_t,
                    idx_u, idx_i, slab, out_u, out_i, *sems):
    wid = lax.axis_index("s") * NC + lax.axis_index("c")
    base = wid * ROWS_PER_W
    iota = lax.iota(jnp.int32, D)

    pltpu.sync_copy(uid.at[pl.ds(base, ROWS_PER_W)], idx_u)
    pltpu.sync_copy(iid.at[pl.ds(base, ROWS_PER_W)], idx_i)

    idxs = (idx_u, idx_i)
    tabs = (uemb_t, iemb_t)
    outs = (out_u, out_i)
    NK = 2 * NG

    def fire(k_par, slot, g):
        idx, tab = idxs[k_par], tabs[k_par]
        iv = idx[pl.ds(g * GRP, GRP)]
        cv = lax.shift_right_logical(iv, 7)
        for b in range(GRP):
            pltpu.async_copy(tab.at[:, pl.ds(cv[b] * LANE, LANE)],
                             slab.at[slot, b], sems[slot])

    def drain(slot):
        for b in range(GRP):
            pltpu.make_async_copy(tabs[0].at[:, pl.ds(0, LANE)],
                                  slab.at[slot, b], sems[slot]).wait()

    def extract(k_par, slot, g):
        idx, out_v = idxs[k_par], outs[k_par]
        iv = idx[pl.ds(g * GRP, GRP)]
        lv = lax.bitwise_and(iv, LANE - 1)
        for b in range(GRP):
            vec = plsc.load_gather(
                slab.at[slot, b], [iota, jnp.full((D,), lv[b], jnp.int32)])
            plsc.store_scatter(
                out_v, [iota, jnp.full((D,), g * GRP + b, jnp.int32)], vec)

    UNROLL = 6
    NBODY = (NK - 4) // UNROLL

    fire(0, 0, 0)
    fire(1, 1, 0)

    def grp_body(j, carry):
        for m in range(UNROLL):
            kf = UNROLL * j + m + 2
            fire(m % 2, (m + 2) % NSLOT, kf // 2)
            drain(m % NSLOT)
            extract(m % 2, m % NSLOT, 3 * j + m // 2)
        return carry

    lax.fori_loop(0, NBODY, grp_body, 0)
    t0 = NK - 4
    fire(0, (t0 + 2) % NSLOT, (t0 + 2) // 2)
    drain(t0 % NSLOT)
    extract(0, t0 % NSLOT, t0 // 2)
    fire(1, (t0 + 3) % NSLOT, (t0 + 3) // 2)
    drain((t0 + 1) % NSLOT)
    extract(1, (t0 + 1) % NSLOT, (t0 + 1) // 2)
    drain((t0 + 2) % NSLOT)
    extract(0, (t0 + 2) % NSLOT, (t0 + 2) // 2)
    drain((t0 + 3) % NSLOT)
    extract(1, (t0 + 3) % NSLOT, (t0 + 3) // 2)
    pltpu.sync_copy(out_u, gu_t.at[:, pl.ds(base, ROWS_PER_W)])
    pltpu.sync_copy(out_i, gi_t.at[:, pl.ds(base, ROWS_PER_W)])


@jax.jit
def _sc_gather(uid, iid, uemb_t, iemb_t):
    mesh = plsc.VectorSubcoreMesh(core_axis_name="c", subcore_axis_name="s")
    return pl.kernel(
        _sc_gather_body,
        out_type=(
            jax.ShapeDtypeStruct((D, B), jnp.float32),
            jax.ShapeDtypeStruct((D, B), jnp.float32),
        ),
        mesh=mesh,
        scratch_types=[
            pltpu.VMEM((ROWS_PER_W,), jnp.int32),
            pltpu.VMEM((ROWS_PER_W,), jnp.int32),
            pltpu.VMEM((NSLOT, GRP, D, LANE), jnp.float32),
            pltpu.VMEM((D, ROWS_PER_W), jnp.float32),
            pltpu.VMEM((D, ROWS_PER_W), jnp.float32),
        ] + [pltpu.SemaphoreType.DMA] * NSLOT,
        compiler_params=pltpu.CompilerParams(needs_layout_passes=False),
    )(uid, iid, uemb_t, iemb_t)


BLK = 4096


def _mlp_body(gu_t, gi_t, w1u, w1i, b1, w2, b2, w3, b3, out):
    dn = (((0,), (0,)), ((), ()))
    h = lax.dot_general(w1u[...], gu_t[...], dn,
                        preferred_element_type=jnp.float32)
    h = h + lax.dot_general(w1i[...], gi_t[...], dn,
                            preferred_element_type=jnp.float32)
    h = jnp.maximum(h + b1[...], 0.0)
    h = jnp.maximum(
        lax.dot_general(w2[...], h, dn, preferred_element_type=jnp.float32)
        + b2[...], 0.0)
    o = lax.dot_general(w3[...], h, dn,
                        preferred_element_type=jnp.float32) + b3[...]
    out[...] = jnp.reshape(o, (BLK,))


def _mlp(gu_t, gi_t, W1, b1, W2, b2, W3, b3):
    w1u = W1[:D, :]
    w1i = W1[D:, :]
    b1r = jnp.reshape(b1, (-1, 1))
    b2r = jnp.reshape(b2, (-1, 1))
    b3r = jnp.reshape(b3, (1, 1))
    grid = (B // BLK,)
    flat = pl.pallas_call(
        _mlp_body,
        grid=grid,
        in_specs=[
            pl.BlockSpec((D, BLK), lambda i: (0, i)),
            pl.BlockSpec((D, BLK), lambda i: (0, i)),
            pl.BlockSpec(w1u.shape, lambda i: (0, 0)),
            pl.BlockSpec(w1i.shape, lambda i: (0, 0)),
            pl.BlockSpec(b1r.shape, lambda i: (0, 0)),
            pl.BlockSpec(W2.shape, lambda i: (0, 0)),
            pl.BlockSpec(b2r.shape, lambda i: (0, 0)),
            pl.BlockSpec(W3.shape, lambda i: (0, 0)),
            pl.BlockSpec(b3r.shape, lambda i: (0, 0)),
        ],
        out_specs=pl.BlockSpec((BLK,), lambda i: (i,)),
        out_shape=jax.ShapeDtypeStruct((B,), jnp.float32),
    )(gu_t, gi_t, w1u, w1i, b1r, W2, b2r, W3, b3r)
    return jnp.reshape(flat, (B, 1))


def kernel(user_id, item_id, user_emb, item_emb, W1, b1, W2, b2, W3, b3):
    gu_t, gi_t = _sc_gather(user_id.astype(jnp.int32),
                            item_id.astype(jnp.int32),
                            user_emb.T, item_emb.T)
    return _mlp(gu_t, gi_t, W1, b1, W2, b2, W3, b3)


# speedup vs baseline: 7.7743x; 1.0004x over previous
import functools

import jax
import jax.numpy as jnp
from jax import lax
from jax.experimental import pallas as pl
from jax.experimental.pallas import tpu as pltpu
from jax.experimental.pallas import tpu_sc as plsc

B = 16384
D = 16
NROWS = 1000000
LANE = 128
NC = 2
NS = 16
NW = NC * NS
ROWS_PER_W = B // NW
GRP = 16
NG = ROWS_PER_W // GRP
NSLOT = 3


def _sc_gather_body(uid, iid, uemb_t, iemb_t, gu_t, gi_t,
                    idx_u, idx_i, slab, out_u, out_i, *sems):
    wid = lax.axis_index("s") * NC + lax.axis_index("c")
    base = wid * ROWS_PER_W
    iota = lax.iota(jnp.int32, D)

    pltpu.sync_copy(uid.at[pl.ds(base, ROWS_PER_W)], idx_u)
    pltpu.sync_copy(iid.at[pl.ds(base, ROWS_PER_W)], idx_i)

    idxs = (idx_u, idx_i)
    tabs = (uemb_t, iemb_t)
    outs = (out_u, out_i)
    NK = 2 * NG

    def fire(k_par, slot, g):
        idx, tab = idxs[k_par], tabs[k_par]
        iv = idx[pl.ds(g * GRP, GRP)]
        cv = lax.shift_right_logical(iv, 7)
        for b in range(GRP):
            pltpu.async_copy(
                tab.at[pl.ds(0, 8), pl.ds(cv[b] * LANE, LANE)],
                slab.at[slot, b, pl.ds(0, 8)], sems[slot])
            pltpu.async_copy(
                tab.at[pl.ds(8, 8), pl.ds(cv[b] * LANE, LANE)],
                slab.at[slot, b, pl.ds(8, 8)], sems[slot])

    def drain(slot):
        for b in range(GRP):
            pltpu.make_async_copy(tabs[0].at[:, pl.ds(0, LANE)],
                                  slab.at[slot, b], sems[slot]).wait()

    def extract(k_par, slot, g):
        idx, out_v = idxs[k_par], outs[k_par]
        iv = idx[pl.ds(g * GRP, GRP)]
        lv = lax.bitwise_and(iv, LANE - 1)
        for b in range(GRP):
            vec = plsc.load_gather(
                slab.at[slot, b], [iota, jnp.full((D,), lv[b], jnp.int32)])
            plsc.store_scatter(
                out_v, [iota, jnp.full((D,), g * GRP + b, jnp.int32)], vec)

    UNROLL = 6
    NBODY = (NK - 4) // UNROLL

    fire(0, 0, 0)
    fire(1, 1, 0)

    def grp_body(j, carry):
        for m in range(UNROLL):
            kf = UNROLL * j + m + 2
            fire(m % 2, (m + 2) % NSLOT, kf // 2)
            drain(m % NSLOT)
            extract(m % 2, m % NSLOT, 3 * j + m // 2)
        return carry

    lax.fori_loop(0, NBODY, grp_body, 0)
    t0 = NK - 4
    fire(0, (t0 + 2) % NSLOT, (t0 + 2) // 2)
    drain(t0 % NSLOT)
    extract(0, t0 % NSLOT, t0 // 2)
    fire(1, (t0 + 3) % NSLOT, (t0 + 3) // 2)
    drain((t0 + 1) % NSLOT)
    extract(1, (t0 + 1) % NSLOT, (t0 + 1) // 2)
    drain((t0 + 2) % NSLOT)
    extract(0, (t0 + 2) % NSLOT, (t0 + 2) // 2)
    drain((t0 + 3) % NSLOT)
    extract(1, (t0 + 3) % NSLOT, (t0 + 3) // 2)
    pltpu.sync_copy(out_u, gu_t.at[:, pl.ds(base, ROWS_PER_W)])
    pltpu.sync_copy(out_i, gi_t.at[:, pl.ds(base, ROWS_PER_W)])


@jax.jit
def _sc_gather(uid, iid, uemb_t, iemb_t):
    mesh = plsc.VectorSubcoreMesh(core_axis_name="c", subcore_axis_name="s")
    return pl.kernel(
        _sc_gather_body,
        out_type=(
            jax.ShapeDtypeStruct((D, B), jnp.float32),
            jax.ShapeDtypeStruct((D, B), jnp.float32),
        ),
        mesh=mesh,
        scratch_types=[
            pltpu.VMEM((ROWS_PER_W,), jnp.int32),
            pltpu.VMEM((ROWS_PER_W,), jnp.int32),
            pltpu.VMEM((NSLOT, GRP, D, LANE), jnp.float32),
            pltpu.VMEM((D, ROWS_PER_W), jnp.float32),
            pltpu.VMEM((D, ROWS_PER_W), jnp.float32),
        ] + [pltpu.SemaphoreType.DMA] * NSLOT,
        compiler_params=pltpu.CompilerParams(needs_layout_passes=False),
    )(uid, iid, uemb_t, iemb_t)


BLK = 4096


def _mlp_body(gu_t, gi_t, w1u, w1i, b1, w2, b2, w3, b3, out):
    dn = (((0,), (0,)), ((), ()))
    h = lax.dot_general(w1u[...], gu_t[...], dn,
                        preferred_element_type=jnp.float32)
    h = h + lax.dot_general(w1i[...], gi_t[...], dn,
                            preferred_element_type=jnp.float32)
    h = jnp.maximum(h + b1[...], 0.0)
    h = jnp.maximum(
        lax.dot_general(w2[...], h, dn, preferred_element_type=jnp.float32)
        + b2[...], 0.0)
    o = lax.dot_general(w3[...], h, dn,
                        preferred_element_type=jnp.float32) + b3[...]
    out[...] = jnp.reshape(o, (BLK,))


def _mlp(gu_t, gi_t, W1, b1, W2, b2, W3, b3):
    w1u = W1[:D, :]
    w1i = W1[D:, :]
    b1r = jnp.reshape(b1, (-1, 1))
    b2r = jnp.reshape(b2, (-1, 1))
    b3r = jnp.reshape(b3, (1, 1))
    grid = (B // BLK,)
    flat = pl.pallas_call(
        _mlp_body,
        grid=grid,
        in_specs=[
            pl.BlockSpec((D, BLK), lambda i: (0, i)),
            pl.BlockSpec((D, BLK), lambda i: (0, i)),
            pl.BlockSpec(w1u.shape, lambda i: (0, 0)),
            pl.BlockSpec(w1i.shape, lambda i: (0, 0)),
            pl.BlockSpec(b1r.shape, lambda i: (0, 0)),
            pl.BlockSpec(W2.shape, lambda i: (0, 0)),
            pl.BlockSpec(b2r.shape, lambda i: (0, 0)),
            pl.BlockSpec(W3.shape, lambda i: (0, 0)),
            pl.BlockSpec(b3r.shape, lambda i: (0, 0)),
        ],
        out_specs=pl.BlockSpec((BLK,), lambda i: (i,)),
        out_shape=jax.ShapeDtypeStruct((B,), jnp.float32),
    )(gu_t, gi_t, w1u, w1i, b1r, W2, b2r, W3, b3r)
    return jnp.reshape(flat, (B, 1))


def kernel(user_id, item_id, user_emb, item_emb, W1, b1, W2, b2, W3, b3):
    gu_t, gi_t = _sc_gather(user_id.astype(jnp.int32),
                            item_id.astype(jnp.int32),
                            user_emb.T, item_emb.T)
    return _mlp(gu_t, gi_t, W1, b1, W2, b2, W3, b3)
